# baseline (speedup 1.0000x reference)
import functools

import jax
import jax.numpy as jnp
from jax import lax
from jax.experimental import pallas as pl
from jax.experimental.pallas import tpu as pltpu
from jax.experimental.pallas import tpu_sc as plsc

_D = 128
_L = 50
_CH = 128


def _sc_gather(table, idx_flat, n_pad):
    info = plsc.get_sparse_core_info()
    nw = info.num_cores * info.num_subcores
    b_per_w = n_pad // nw
    n_ch = b_per_w // _CH
    assert n_ch >= 8 and n_ch % 4 == 0
    mesh = plsc.VectorSubcoreMesh(core_axis_name="c", subcore_axis_name="s")

    @functools.partial(
        pl.kernel,
        out_type=jax.ShapeDtypeStruct((n_pad, _D), jnp.float32),
        mesh=mesh,
        scratch_types=[
            pltpu.VMEM((b_per_w,), jnp.int32),
            pltpu.VMEM((4, _CH, _D), jnp.float32),
            [pltpu.SemaphoreType.DMA] * 4,
            [pltpu.SemaphoreType.DMA] * 4,
        ],
    )
    def k(table_hbm, idx_hbm, out_hbm, idx_v, rows_v, sg, sw):
        wid = lax.axis_index("s") * info.num_cores + lax.axis_index("c")
        base = wid * b_per_w
        pltpu.sync_copy(idx_hbm.at[pl.ds(base, b_per_w)], idx_v)

        def g_copy(c, buf):
            return pltpu.make_async_copy(
                table_hbm.at[idx_v.at[pl.ds(c * _CH, _CH)]],
                rows_v.at[buf], sg[buf])

        def w_copy(c, buf):
            return pltpu.make_async_copy(
                rows_v.at[buf], out_hbm.at[pl.ds(base + c * _CH, _CH)],
                sw[buf])

        g_copy(0, 0).start()
        g_copy(1, 1).start()
        g_copy(2, 2).start()
        g_copy(0, 0).wait()
        w_copy(0, 0).start()
        g_copy(3, 3).start()

        def body(q, carry):
            for kq in range(4):
                c = 1 + 4 * q + kq
                b = (1 + kq) % 4
                pb = kq % 4
                g_copy(c, b).wait()
                w_copy(c, b).start()
                w_copy(c - 1, pb).wait()
                g_copy(c + 3, pb).start()
            return carry

        lax.fori_loop(0, (n_ch - 4) // 4, body, 0)

        for c in (n_ch - 3, n_ch - 2, n_ch - 1):
            b = c % 4
            g_copy(c, b).wait()
            w_copy(c, b).start()
        for c in (n_ch - 4, n_ch - 3, n_ch - 2, n_ch - 1):
            w_copy(c, c % 4).wait()

    return k(table, idx_flat)


def _tc_body(mail_ref, dst_ref, t_ref, te_ref, tek_ref, w1_ref, b1_ref,
             w2_ref, b2_ref, out_ref):
    mail = mail_ref[...]
    dst = dst_ref[...]
    t = t_ref[...]
    inv_scale = float(1.0 / (128.0 ** 0.5))

    tl = t[:, None, :]
    tm = t[:, :, None]
    li = lax.broadcasted_iota(jnp.int32, (1, _L, _L), 2)
    mi = lax.broadcasted_iota(jnp.int32, (1, _L, _L), 1)
    cmp = (tm < tl) | ((tm == tl) & (mi < li))
    order = jnp.sum(cmp.astype(jnp.int32), axis=1)
    re_order = (_L - 1) - order

    tedot = lax.dot_general(dst, te_ref[...],
                            (((1,), (1,)), ((), ())))
    ji = lax.broadcasted_iota(jnp.int32, (1, _L, _L), 2)
    ohf = (re_order[:, :, None] == ji).astype(jnp.float32)
    e_te = jnp.sum(ohf * tedot[:, None, :], axis=2)
    s = jnp.sum(mail * dst[:, None, :], axis=2)
    e = (e_te + s) * inv_scale
    e = e - jnp.max(e, axis=1, keepdims=True)
    ex = jnp.exp(e)
    alpha = ex * (1.0 / jnp.sum(ex, axis=1, keepdims=True))

    beta = jnp.sum(ohf * alpha[:, :, None], axis=1)
    tek_term = jnp.dot(beta, tek_ref[...])

    tmax = jnp.max(t, axis=1, keepdims=True)
    l1 = lax.broadcasted_iota(jnp.int32, (1, _L), 1)
    lsel = jnp.min(jnp.where(t == tmax, l1, _L), axis=1, keepdims=True)
    last_oh = (l1 == lsel).astype(jnp.float32)
    last_em = jnp.sum(last_oh[:, :, None] * mail, axis=1)
    e1 = jnp.sum(mail * last_em[:, None, :], axis=2) * inv_scale
    e1 = e1 - jnp.max(e1, axis=1, keepdims=True)
    ex1 = jnp.exp(e1)
    alpha1 = ex1 * (1.0 / jnp.sum(ex1, axis=1, keepdims=True))

    w = alpha + alpha1
    hmail = jnp.sum(w[:, :, None] * mail, axis=1)
    h = hmail + tek_term

    z = jnp.maximum(jnp.dot(h, w1_ref[...]) + b1_ref[...], 0.0)
    z = jnp.dot(z, w2_ref[...]) + b2_ref[...] + dst
    out_ref[...] = jnp.where(z > 0.0, z, jnp.exp(z) - 1.0)


def _tc_side(mail, dst_h, times, te, tek, w1, b1, w2, b2, block_b=200,
             interpret=False):
    n = dst_h.shape[0]
    grid = n // block_b
    return pl.pallas_call(
        _tc_body,
        grid=(grid,),
        in_specs=[
            pl.BlockSpec((block_b, _L, _D), lambda i: (i, 0, 0)),
            pl.BlockSpec((block_b, _D), lambda i: (i, 0)),
            pl.BlockSpec((block_b, _L), lambda i: (i, 0)),
            pl.BlockSpec((_L, _D), lambda i: (0, 0)),
            pl.BlockSpec((_L, _D), lambda i: (0, 0)),
            pl.BlockSpec((_D, _D), lambda i: (0, 0)),
            pl.BlockSpec((1, _D), lambda i: (0, 0)),
            pl.BlockSpec((_D, _D), lambda i: (0, 0)),
            pl.BlockSpec((1, _D), lambda i: (0, 0)),
        ],
        out_specs=pl.BlockSpec((block_b, _D), lambda i: (i, 0)),
        out_shape=jax.ShapeDtypeStruct((n, _D), jnp.float32),
        interpret=interpret,
    )(mail, dst_h, times, te, tek, w1, b1.reshape(1, _D), w2,
      b2.reshape(1, _D))


def kernel(user_h, item_h, i_time_enc, i_time_enc_k, u_time_enc,
           u_time_enc_k, gu_W1, gu_b1, gu_W2, gu_b2, gi_W1, gi_b1, gi_W2,
           gi_b2, u_neighbors, i_neighbors, u_times, i_times):
    n_item, l = i_neighbors.shape
    n_user = u_neighbors.shape[0]
    info = plsc.get_sparse_core_info()
    nw = info.num_cores * info.num_subcores
    step = nw * _CH

    def gather_side(table, idx):
        n_rows = idx.shape[0] * l
        step2 = 4 * step
        n_pad = ((n_rows + step2 - 1) // step2) * step2
        idx_flat = jnp.concatenate(
            [idx.reshape(-1),
             jnp.zeros((n_pad - n_rows,), dtype=jnp.int32)])
        rows = _sc_gather(table, idx_flat, n_pad)
        return rows[:n_rows].reshape(idx.shape[0], l, _D)

    half_i = n_item // 2
    half_u = n_user // 2
    segs = [
        (user_h, i_neighbors[:half_i], item_h[:half_i], i_times[:half_i],
         i_time_enc, i_time_enc_k, gi_W1, gi_b1, gi_W2, gi_b2),
        (user_h, i_neighbors[half_i:], item_h[half_i:], i_times[half_i:],
         i_time_enc, i_time_enc_k, gi_W1, gi_b1, gi_W2, gi_b2),
        (item_h, u_neighbors[:half_u], user_h[:half_u], u_times[:half_u],
         u_time_enc, u_time_enc_k, gu_W1, gu_b1, gu_W2, gu_b2),
        (item_h, u_neighbors[half_u:], user_h[half_u:], u_times[half_u:],
         u_time_enc, u_time_enc_k, gu_W1, gu_b1, gu_W2, gu_b2),
    ]
    mails = [gather_side(sg[0], sg[1]) for sg in segs]
    outs = [
        _tc_side(m, sg[2], sg[3], sg[4], sg[5], sg[6], sg[7], sg[8], sg[9])
        for m, sg in zip(mails, segs)
    ]
    item_new = jnp.concatenate([outs[0], outs[1]], axis=0)
    user_new = jnp.concatenate([outs[2], outs[3]], axis=0)
    return (user_new, item_new)


# speedup vs baseline: 1.3300x; 1.3300x over previous
import functools

import jax
import jax.numpy as jnp
from jax import lax
from jax.experimental import pallas as pl
from jax.experimental.pallas import tpu as pltpu
from jax.experimental.pallas import tpu_sc as plsc

_D = 128
_L = 50
_CH = 128


def _sc_gather2(table_a, idx_a, table_b, idx_b, n_pad):
    info = plsc.get_sparse_core_info()
    nw = info.num_cores * info.num_subcores
    b_per_w = n_pad // nw
    n_ch = b_per_w // _CH
    assert n_ch >= 8 and n_ch % 4 == 0
    mesh = plsc.VectorSubcoreMesh(core_axis_name="c", subcore_axis_name="s")

    @functools.partial(
        pl.kernel,
        out_type=(jax.ShapeDtypeStruct((n_pad, _D), jnp.float32),
                  jax.ShapeDtypeStruct((n_pad, _D), jnp.float32)),
        mesh=mesh,
        scratch_types=[
            pltpu.VMEM((b_per_w,), jnp.int32),
            pltpu.VMEM((4, _CH, _D), jnp.float32),
            [pltpu.SemaphoreType.DMA] * 4,
            [pltpu.SemaphoreType.DMA] * 4,
        ],
    )
    def k(ta_hbm, ia_hbm, tb_hbm, ib_hbm, oa_hbm, ob_hbm, idx_v, rows_v,
          sg, sw):
        wid = lax.axis_index("s") * info.num_cores + lax.axis_index("c")
        base = wid * b_per_w

        def ring(table_hbm, idx_hbm, out_hbm):
            pltpu.sync_copy(idx_hbm.at[pl.ds(base, b_per_w)], idx_v)

            def g_copy(c, buf):
                return pltpu.make_async_copy(
                    table_hbm.at[idx_v.at[pl.ds(c * _CH, _CH)]],
                    rows_v.at[buf], sg[buf])

            def w_copy(c, buf):
                return pltpu.make_async_copy(
                    rows_v.at[buf], out_hbm.at[pl.ds(base + c * _CH, _CH)],
                    sw[buf])

            g_copy(0, 0).start()
            g_copy(1, 1).start()
            g_copy(2, 2).start()
            g_copy(0, 0).wait()
            w_copy(0, 0).start()
            g_copy(3, 3).start()

            def body(q, carry):
                for kq in range(4):
                    c = 1 + 4 * q + kq
                    b = (1 + kq) % 4
                    pb = kq % 4
                    g_copy(c, b).wait()
                    w_copy(c, b).start()
                    w_copy(c - 1, pb).wait()
                    g_copy(c + 3, pb).start()
                return carry

            lax.fori_loop(0, (n_ch - 4) // 4, body, 0)

            for c in (n_ch - 3, n_ch - 2, n_ch - 1):
                b = c % 4
                g_copy(c, b).wait()
                w_copy(c, b).start()
            for c in (n_ch - 4, n_ch - 3, n_ch - 2, n_ch - 1):
                w_copy(c, c % 4).wait()

        ring(ta_hbm, ia_hbm, oa_hbm)
        ring(tb_hbm, ib_hbm, ob_hbm)

    return k(table_a, idx_a, table_b, idx_b)


def _tc_body(mail_ref, dst_ref, t_ref, te_ref, tek_ref, w1_ref, b1_ref,
             w2_ref, b2_ref, out_ref):
    mail = mail_ref[...]
    dst = dst_ref[...]
    t = t_ref[...]
    inv_scale = float(1.0 / (128.0 ** 0.5))

    tl = t[:, None, :]
    tm = t[:, :, None]
    li = lax.broadcasted_iota(jnp.int32, (1, _L, _L), 2)
    mi = lax.broadcasted_iota(jnp.int32, (1, _L, _L), 1)
    cmp = (tm < tl) | ((tm == tl) & (mi < li))
    order = jnp.sum(cmp.astype(jnp.int32), axis=1)
    re_order = (_L - 1) - order

    tedot = lax.dot_general(dst, te_ref[...],
                            (((1,), (1,)), ((), ())))
    ji = lax.broadcasted_iota(jnp.int32, (1, _L, _L), 2)
    ohf = (re_order[:, :, None] == ji).astype(jnp.float32)
    e_te = jnp.sum(ohf * tedot[:, None, :], axis=2)
    s = jnp.sum(mail * dst[:, None, :], axis=2)
    e = (e_te + s) * inv_scale
    e = e - jnp.max(e, axis=1, keepdims=True)
    ex = jnp.exp(e)
    alpha = ex * (1.0 / jnp.sum(ex, axis=1, keepdims=True))

    beta = jnp.sum(ohf * alpha[:, :, None], axis=1)
    tek_term = jnp.dot(beta, tek_ref[...])

    tmax = jnp.max(t, axis=1, keepdims=True)
    l1 = lax.broadcasted_iota(jnp.int32, (1, _L), 1)
    lsel = jnp.min(jnp.where(t == tmax, l1, _L), axis=1, keepdims=True)
    last_oh = (l1 == lsel).astype(jnp.float32)
    last_em = jnp.sum(last_oh[:, :, None] * mail, axis=1)
    e1 = jnp.sum(mail * last_em[:, None, :], axis=2) * inv_scale
    e1 = e1 - jnp.max(e1, axis=1, keepdims=True)
    ex1 = jnp.exp(e1)
    alpha1 = ex1 * (1.0 / jnp.sum(ex1, axis=1, keepdims=True))

    w = alpha + alpha1
    hmail = jnp.sum(w[:, :, None] * mail, axis=1)
    h = hmail + tek_term

    z = jnp.maximum(jnp.dot(h, w1_ref[...]) + b1_ref[...], 0.0)
    z = jnp.dot(z, w2_ref[...]) + b2_ref[...] + dst
    out_ref[...] = jnp.where(z > 0.0, z, jnp.exp(z) - 1.0)


def _tc_side(mail, dst_h, times, te, tek, w1, b1, w2, b2, block_b=200,
             interpret=False):
    n = dst_h.shape[0]
    grid = n // block_b
    return pl.pallas_call(
        _tc_body,
        grid=(grid,),
        in_specs=[
            pl.BlockSpec((block_b, _L, _D), lambda i: (i, 0, 0)),
            pl.BlockSpec((block_b, _D), lambda i: (i, 0)),
            pl.BlockSpec((block_b, _L), lambda i: (i, 0)),
            pl.BlockSpec((_L, _D), lambda i: (0, 0)),
            pl.BlockSpec((_L, _D), lambda i: (0, 0)),
            pl.BlockSpec((_D, _D), lambda i: (0, 0)),
            pl.BlockSpec((1, _D), lambda i: (0, 0)),
            pl.BlockSpec((_D, _D), lambda i: (0, 0)),
            pl.BlockSpec((1, _D), lambda i: (0, 0)),
        ],
        out_specs=pl.BlockSpec((block_b, _D), lambda i: (i, 0)),
        out_shape=jax.ShapeDtypeStruct((n, _D), jnp.float32),
        interpret=interpret,
    )(mail, dst_h, times, te, tek, w1, b1.reshape(1, _D), w2,
      b2.reshape(1, _D))


def kernel(user_h, item_h, i_time_enc, i_time_enc_k, u_time_enc,
           u_time_enc_k, gu_W1, gu_b1, gu_W2, gu_b2, gi_W1, gi_b1, gi_W2,
           gi_b2, u_neighbors, i_neighbors, u_times, i_times):
    n_item, l = i_neighbors.shape
    n_user = u_neighbors.shape[0]
    info = plsc.get_sparse_core_info()
    nw = info.num_cores * info.num_subcores
    step = nw * _CH

    n_rows = n_item * l
    step2 = 4 * step
    n_pad = ((n_rows + step2 - 1) // step2) * step2
    pad = jnp.zeros((n_pad - n_rows,), dtype=jnp.int32)
    idx_i = jnp.concatenate([i_neighbors.reshape(-1), pad])
    idx_u = jnp.concatenate([u_neighbors.reshape(-1), pad])
    rows_i, rows_u = _sc_gather2(user_h, idx_i, item_h, idx_u, n_pad)
    mail_for_item = rows_i[:n_rows].reshape(n_item, l, _D)
    mail_for_user = rows_u[:n_rows].reshape(n_user, l, _D)

    item_new = _tc_side(mail_for_item, item_h, i_times, i_time_enc,
                        i_time_enc_k, gi_W1, gi_b1, gi_W2, gi_b2)
    user_new = _tc_side(mail_for_user, user_h, u_times, u_time_enc,
                        u_time_enc_k, gu_W1, gu_b1, gu_W2, gu_b2)
    return (user_new, item_new)


# speedup vs baseline: 1.5404x; 1.1581x over previous
import functools

import jax
import jax.numpy as jnp
from jax import lax
from jax.experimental import pallas as pl
from jax.experimental.pallas import tpu as pltpu
from jax.experimental.pallas import tpu_sc as plsc

_D = 128
_L = 50
_CH = 64
_NBUF = 8


def _sc_gather(table, idx_flat, n_pad):
    info = plsc.get_sparse_core_info()
    nw = info.num_cores * info.num_subcores
    b_per_w = n_pad // nw
    n_ch = b_per_w // _CH
    assert n_ch >= 2 * _NBUF and n_ch % _NBUF == 0
    mesh = plsc.VectorSubcoreMesh(core_axis_name="c", subcore_axis_name="s")

    @functools.partial(
        pl.kernel,
        out_type=jax.ShapeDtypeStruct((n_pad, _D), jnp.float32),
        mesh=mesh,
        scratch_types=[
            pltpu.VMEM((b_per_w,), jnp.int32),
            pltpu.VMEM((_NBUF, _CH, _D), jnp.float32),
            [pltpu.SemaphoreType.DMA] * _NBUF,
            [pltpu.SemaphoreType.DMA] * _NBUF,
        ],
    )
    def k(table_hbm, idx_hbm, out_hbm, idx_v, rows_v, sg, sw):
        wid = lax.axis_index("s") * info.num_cores + lax.axis_index("c")
        base = wid * b_per_w
        if True:
            pltpu.sync_copy(idx_hbm.at[pl.ds(base, b_per_w)], idx_v)

            def g_copy(c, buf):
                return pltpu.make_async_copy(
                    table_hbm.at[idx_v.at[pl.ds(c * _CH, _CH)]],
                    rows_v.at[buf], sg[buf])

            def w_copy(c, buf):
                return pltpu.make_async_copy(
                    rows_v.at[buf], out_hbm.at[pl.ds(base + c * _CH, _CH)],
                    sw[buf])

            for j in range(_NBUF - 1):
                g_copy(j, j).start()
            g_copy(0, 0).wait()
            w_copy(0, 0).start()
            g_copy(_NBUF - 1, _NBUF - 1).start()

            def body(q, carry):
                for kq in range(_NBUF):
                    c = 1 + _NBUF * q + kq
                    b = (1 + kq) % _NBUF
                    pb = kq % _NBUF
                    g_copy(c, b).wait()
                    w_copy(c, b).start()
                    w_copy(c - 1, pb).wait()
                    g_copy(c + _NBUF - 1, pb).start()
                return carry

            lax.fori_loop(0, (n_ch - _NBUF) // _NBUF, body, 0)

            for c in range(n_ch - _NBUF + 1, n_ch):
                g_copy(c, c % _NBUF).wait()
                w_copy(c, c % _NBUF).start()
            for c in range(n_ch - _NBUF, n_ch):
                w_copy(c, c % _NBUF).wait()

    return k(table, idx_flat)


def _tc_body(mail_ref, dst_ref, t_ref, te_ref, tek_ref, w1_ref, b1_ref,
             w2_ref, b2_ref, out_ref):
    mail = mail_ref[...]
    dst = dst_ref[...]
    t = t_ref[...]
    inv_scale = float(1.0 / (128.0 ** 0.5))

    key = t * 64 + lax.broadcasted_iota(jnp.int32, (1, _L), 1)
    kl = key[:, None, :]
    km = key[:, :, None]
    cmp = km < kl
    order = jnp.sum(cmp.astype(jnp.int32), axis=1)
    re_order = (_L - 1) - order

    tedot = lax.dot_general(dst, te_ref[...],
                            (((1,), (1,)), ((), ())))
    ji = lax.broadcasted_iota(jnp.int32, (1, _L, _L), 2)
    ohf = (re_order[:, :, None] == ji).astype(jnp.float32)
    e_te = jnp.sum(ohf * tedot[:, None, :], axis=2)
    s = jnp.sum(mail * dst[:, None, :], axis=2)
    e = (e_te + s) * inv_scale
    ex = jnp.exp(e)
    alpha = ex * (1.0 / jnp.sum(ex, axis=1, keepdims=True))

    beta = jnp.sum(ohf * alpha[:, :, None], axis=1)
    tek_term = jnp.dot(beta, tek_ref[...])

    tmax = jnp.max(t, axis=1, keepdims=True)
    l1 = lax.broadcasted_iota(jnp.int32, (1, _L), 1)
    lsel = jnp.min(jnp.where(t == tmax, l1, _L), axis=1, keepdims=True)
    last_oh = (l1 == lsel).astype(jnp.float32)
    last_em = jnp.sum(last_oh[:, :, None] * mail, axis=1)
    e1 = jnp.sum(mail * last_em[:, None, :], axis=2) * inv_scale
    ex1 = jnp.exp(e1)
    alpha1 = ex1 * (1.0 / jnp.sum(ex1, axis=1, keepdims=True))

    w = alpha + alpha1
    hmail = jnp.sum(w[:, :, None] * mail, axis=1)
    h = hmail + tek_term

    z = jnp.maximum(jnp.dot(h, w1_ref[...]) + b1_ref[...], 0.0)
    z = jnp.dot(z, w2_ref[...]) + b2_ref[...] + dst
    out_ref[...] = jnp.where(z > 0.0, z, jnp.exp(z) - 1.0)


def _tc_side(mail, dst_h, times, te, tek, w1, b1, w2, b2, block_b=200,
             interpret=False):
    n = dst_h.shape[0]
    grid = n // block_b
    return pl.pallas_call(
        _tc_body,
        grid=(grid,),
        in_specs=[
            pl.BlockSpec((block_b, _L, _D), lambda i: (i, 0, 0)),
            pl.BlockSpec((block_b, _D), lambda i: (i, 0)),
            pl.BlockSpec((block_b, _L), lambda i: (i, 0)),
            pl.BlockSpec((_L, _D), lambda i: (0, 0)),
            pl.BlockSpec((_L, _D), lambda i: (0, 0)),
            pl.BlockSpec((_D, _D), lambda i: (0, 0)),
            pl.BlockSpec((1, _D), lambda i: (0, 0)),
            pl.BlockSpec((_D, _D), lambda i: (0, 0)),
            pl.BlockSpec((1, _D), lambda i: (0, 0)),
        ],
        out_specs=pl.BlockSpec((block_b, _D), lambda i: (i, 0)),
        out_shape=jax.ShapeDtypeStruct((n, _D), jnp.float32),
        interpret=interpret,
    )(mail, dst_h, times, te, tek, w1, b1.reshape(1, _D), w2,
      b2.reshape(1, _D))


def kernel(user_h, item_h, i_time_enc, i_time_enc_k, u_time_enc,
           u_time_enc_k, gu_W1, gu_b1, gu_W2, gu_b2, gi_W1, gi_b1, gi_W2,
           gi_b2, u_neighbors, i_neighbors, u_times, i_times):
    n_item, l = i_neighbors.shape
    n_user = u_neighbors.shape[0]
    info = plsc.get_sparse_core_info()
    nw = info.num_cores * info.num_subcores
    step = nw * _CH

    n_rows = n_item * l
    step2 = _NBUF * step
    n_pad = ((n_rows + step2 - 1) // step2) * step2
    pad = jnp.zeros((n_pad - n_rows,), dtype=jnp.int32)
    idx_i = jnp.concatenate([i_neighbors.reshape(-1), pad])
    idx_u = jnp.concatenate([u_neighbors.reshape(-1), pad])
    rows_i = _sc_gather(user_h, idx_i, n_pad)
    rows_u = _sc_gather(item_h, idx_u, n_pad)
    mail_for_item = rows_i[:n_rows].reshape(n_item, l, _D)
    mail_for_user = rows_u[:n_rows].reshape(n_user, l, _D)

    item_new = _tc_side(mail_for_item, item_h, i_times, i_time_enc,
                        i_time_enc_k, gi_W1, gi_b1, gi_W2, gi_b2)
    user_new = _tc_side(mail_for_user, user_h, u_times, u_time_enc,
                        u_time_enc_k, gu_W1, gu_b1, gu_W2, gu_b2)
    return (user_new, item_new)
